# baseline (speedup 1.0000x reference)
import jax
import jax.numpy as jnp
from jax import lax
from jax.experimental import pallas as pl

_DATA_DIM = 26
_DEPTH = 1000
_BATCH = 1024
_ROWS_PER_BLOCK = 128


def _onehot_block(x_ref, o_ref):
    r = o_ref.shape[0]
    iota = lax.broadcasted_iota(jnp.int32, (r, _DATA_DIM, _DEPTH), 2)
    o_ref[...] = (iota == x_ref[...][:, :, None]).astype(jnp.float32)


def kernel(x):
    grid = (_BATCH // _ROWS_PER_BLOCK,)
    oh = pl.pallas_call(
        _onehot_block,
        grid=grid,
        in_specs=[
            pl.BlockSpec((_ROWS_PER_BLOCK, _DATA_DIM), lambda i: (i, 0)),
        ],
        out_specs=pl.BlockSpec(
            (_ROWS_PER_BLOCK, _DATA_DIM, _DEPTH), lambda i: (i, 0, 0)
        ),
        out_shape=jax.ShapeDtypeStruct(
            (_BATCH, _DATA_DIM, _DEPTH), jnp.float32
        ),
    )(x)
    return oh.reshape(_BATCH, _DATA_DIM * _DEPTH)


# speedup vs baseline: 1.2293x; 1.2293x over previous
import jax
import jax.numpy as jnp
from jax import lax
from jax.experimental import pallas as pl

_DATA_DIM = 26
_DEPTH = 1000
_BATCH = 1024
_ROWS_PER_BLOCK = 128


def _onehot_block(x_ref, o_ref):
    r = o_ref.shape[0]
    iota = lax.broadcasted_iota(jnp.int32, (r, _DEPTH), 1)
    for d in range(_DATA_DIM):
        xd = x_ref[:, d : d + 1]
        o_ref[:, d * _DEPTH : (d + 1) * _DEPTH] = (iota == xd).astype(
            jnp.float32
        )


def kernel(x):
    grid = (_BATCH // _ROWS_PER_BLOCK,)
    return pl.pallas_call(
        _onehot_block,
        grid=grid,
        in_specs=[
            pl.BlockSpec((_ROWS_PER_BLOCK, _DATA_DIM), lambda i: (i, 0)),
        ],
        out_specs=pl.BlockSpec(
            (_ROWS_PER_BLOCK, _DATA_DIM * _DEPTH), lambda i: (i, 0)
        ),
        out_shape=jax.ShapeDtypeStruct(
            (_BATCH, _DATA_DIM * _DEPTH), jnp.float32
        ),
    )(x)
